# baseline (speedup 1.0000x reference)
import functools

import jax
import jax.numpy as jnp
from jax import lax
from jax.experimental import pallas as pl
from jax.experimental.pallas import tpu as pltpu
from jax.experimental.pallas import tpu_sc as plsc

N = 10000
D = 128
H1, C1 = 8, 8
F1 = H1 * C1
NCLS = 16
NACC = 10240
NC, NS, L = 2, 16, 16
NW = NC * NS
T = 10752
EP = NW * T
RPT = NACC // NS

G1, HC1 = 48, 32
GA1 = 80
G2, HC2 = 32, 16

PERM = []
for _g in range(2):
    PERM += [32 * _g + 2 * _l for _l in range(16)]
    PERM += [32 * _g + 2 * _l + 1 for _l in range(16)]


def _edge_kernel(G, HC, He, ad_full, K, NCH, GA, pack_bf16):
    mesh = plsc.VectorSubcoreMesh(core_axis_name="c", subcore_axis_name="s",
                                  num_cores=NC, num_subcores=NS)
    scratch = [
        pltpu.VMEM_SHARED((NACC, GA), jnp.float32),
        pltpu.VMEM((NCH, K), jnp.int32),
        pltpu.VMEM((NCH, K), jnp.int32),
        pltpu.VMEM((NACC,), jnp.float32) if ad_full
        else pltpu.VMEM((2, K, He), jnp.float32),
        pltpu.VMEM((2, K, G), jnp.float32),
        pltpu.VMEM((K * He,), jnp.float32),
        pltpu.VMEM((2, K, GA), jnp.float32),
        pltpu.SemaphoreType.DMA,
        pltpu.SemaphoreType.DMA,
        pltpu.SemaphoreType.DMA,
        pltpu.SemaphoreType.DMA,
        pltpu.SemaphoreType.DMA,
        pltpu.SemaphoreType.DMA,
    ]

    @functools.partial(
        pl.kernel,
        out_type=jax.ShapeDtypeStruct((NC, NACC, GA), jnp.float32),
        mesh=mesh,
        scratch_types=scratch,
        compiler_params=pltpu.CompilerParams(
            use_tc_tiling_on_sc=False, needs_layout_passes=False),
    )
    def body(hs_hbm, ad_hbm, src_hbm, dst_hbm, out_hbm,
             acc, src_t, dst_t, ad_v, rows_v, w_v, ob_v,
             sg0, sg1, sa0, sa1, ss0, ss1):
        c = lax.axis_index("c")
        s = lax.axis_index("s")
        for zp in (0, 1):
            zrows = ob_v.at[zp]

            def zfill(t, cz, zrows=zrows):
                for j in range(GA // L):
                    zrows[t, pl.ds(j * L, L)] = jnp.zeros((L,), jnp.float32)
                return cz

            lax.fori_loop(0, K, zfill, 0, unroll=4)
        zrows = ob_v.at[0]
        nfull = RPT // K
        for r in range(nfull):
            pltpu.sync_copy(zrows, acc.at[pl.ds(s * RPT + r * K, K)])
        if RPT % K:
            pltpu.sync_copy(zrows.at[pl.ds(0, RPT % K)],
                            acc.at[pl.ds(s * RPT + nfull * K, RPT % K)])
        row0 = (c * NS + s) * NCH
        pltpu.sync_copy(src_hbm.at[pl.ds(row0, NCH)], src_t)
        pltpu.sync_copy(dst_hbm.at[pl.ds(row0, NCH)], dst_t)
        if ad_full:
            pltpu.sync_copy(ad_hbm, ad_v)
        plsc.subcore_barrier()
        iota = lax.iota(jnp.int32, L)
        sg = (sg0, sg1)
        sa = (sa0, sa1)
        ss = (ss0, ss1)

        def g_start(i, p):
            pltpu.async_copy(hs_hbm.at[src_t.at[i]], rows_v.at[p], sg[p])
            if not ad_full:
                pltpu.async_copy(ad_hbm.at[dst_t.at[i]], ad_v.at[p], sa[p])

        def g_wait(p):
            pltpu.make_async_copy(hs_hbm.at[src_t.at[0]], rows_v.at[p],
                                  sg[p]).wait()
            if not ad_full:
                pltpu.make_async_copy(ad_hbm.at[dst_t.at[0]], ad_v.at[p],
                                      sa[p]).wait()

        def s_start(i, p):
            pltpu.async_copy(ob_v.at[p], acc.at[dst_t.at[i]], ss[p], add=True)

        def s_wait(p):
            pltpu.make_async_copy(ob_v.at[p], acc.at[dst_t.at[0]],
                                  ss[p]).wait()

        def compute(i, p):
            rows = rows_v.at[p]
            ob = ob_v.at[p]

            def wpass(t, cw):
                p0 = t * L
                pp = p0 + iota
                if He == 1:
                    k_vec = pp
                    h_vec = jnp.zeros((L,), jnp.int32)
                else:
                    k_vec = jnp.right_shift(pp, 3)
                    h_vec = jnp.bitwise_and(pp, He - 1)
                as_vals = plsc.load_gather(rows, [k_vec, HC + h_vec])
                if ad_full:
                    dvals = dst_t[i, pl.ds(p0, L)]
                    ad_vals = plsc.load_gather(ad_v, [dvals])
                else:
                    ad_vals = plsc.load_gather(ad_v.at[p], [k_vec, h_vec])
                e = as_vals + ad_vals
                e = jnp.where(e >= 0.0, e, 0.2 * e)
                w_v[pl.ds(p0, L)] = jnp.exp(e)
                return cw

            lax.fori_loop(0, K * He // L, wpass, 0, unroll=2)

            if pack_bf16:
                def mpass(k, cm):
                    wbase = k * He
                    for g in range(HC // L):
                        v = rows[k, pl.ds(g * L, L)]
                        ab = plsc.bitcast(v, jnp.bfloat16)
                        a, b = plsc.unpack(
                            ab, format=plsc.PackFormat.INTERLEAVED)
                        head = 4 * g + jnp.right_shift(iota, 2)
                        wv = plsc.load_gather(w_v, [wbase + head])
                        ob[k, pl.ds(2 * g * L, L)] = a * wv
                        ob[k, pl.ds((2 * g + 1) * L, L)] = b * wv
                    widx = wbase + jnp.minimum(iota, He - 1)
                    wvals = plsc.load_gather(w_v, [widx])
                    ob[k, pl.ds(2 * HC, L)] = jnp.where(iota < He, wvals, 0.0)
                    return cm
            else:
                def mpass(k, cm):
                    wbase = k * He
                    for j in range(GA // L):
                        if (j + 1) * L <= HC:
                            hv = rows[k, pl.ds(j * L, L)]
                            if He == 1:
                                kvec = jnp.broadcast_to(k, (L,)).astype(jnp.int32)
                                wvals = plsc.load_gather(w_v, [kvec])
                            else:
                                head = jnp.right_shift(j * L + iota, 3)
                                wvals = plsc.load_gather(w_v, [wbase + head])
                            ob[k, pl.ds(j * L, L)] = hv * wvals
                        elif j * L == HC:
                            widx = wbase + jnp.minimum(iota, He - 1)
                            wvals = plsc.load_gather(w_v, [widx])
                            ob[k, pl.ds(j * L, L)] = jnp.where(
                                iota < He, wvals, 0.0)
                    return cm

            lax.fori_loop(0, K, mpass, 0, unroll=4)

        g_start(0, 0)
        g_start(1, 1)
        g_wait(0)
        compute(0, 0)
        g_start(2, 0)
        s_start(0, 0)
        g_wait(1)
        compute(1, 1)
        g_start(3, 1)
        s_start(1, 1)

        def step(i2, carry):
            for p in (0, 1):
                i = 2 * i2 + p
                g_wait(p)
                s_wait(p)
                compute(i, p)
                g_start(jnp.minimum(i + 2, NCH - 1), p)
                s_start(i, p)
            return carry

        lax.fori_loop(1, NCH // 2, step, 0)
        g_wait(0)
        g_wait(1)
        s_wait(0)
        s_wait(1)
        plsc.subcore_barrier()
        off = 0
        while off < RPT:
            blk = min(K, RPT - off)
            tmp = ob_v.at[1, pl.ds(0, blk)]
            pltpu.sync_copy(acc.at[pl.ds(s * RPT + off, blk)], tmp)
            pltpu.sync_copy(tmp, out_hbm.at[c, pl.ds(s * RPT + off, blk)])
            off += blk

    return body


_edge_l1 = _edge_kernel(G1, HC1, H1, False, 128, 84, GA1, True)
_edge_l2 = _edge_kernel(G2, HC2, 1, True, 256, 42, G2, False)


def _tc1_body(x_ref, w_ref, aS_ref, aD_ref, o_ref):
    h = jnp.dot(x_ref[...], w_ref[...], preferred_element_type=jnp.float32)
    aS = jnp.dot(h, aS_ref[...], preferred_element_type=jnp.float32)
    aD = jnp.dot(h, aD_ref[...], preferred_element_type=jnp.float32)
    o_ref[...] = jnp.concatenate([h, aS, aD], axis=1)


_tc1 = pl.pallas_call(
    _tc1_body,
    grid=(10,),
    in_specs=[pl.BlockSpec((N // 10, D), lambda i: (i, 0)),
              pl.BlockSpec((D, F1), lambda i: (0, 0)),
              pl.BlockSpec((F1, H1), lambda i: (0, 0)),
              pl.BlockSpec((F1, H1), lambda i: (0, 0))],
    out_specs=pl.BlockSpec((N // 10, F1 + 2 * H1), lambda i: (i, 0)),
    out_shape=jax.ShapeDtypeStruct((N, F1 + 2 * H1), jnp.float32),
)


def _tc2_body(p1_ref, p2_ref, e8_ref, b1_ref, w2_ref, asd_ref, o_ref):
    acc = p1_ref[...] + p2_ref[...]
    den = jnp.dot(acc[:, F1:F1 + H1], e8_ref[...],
                  preferred_element_type=jnp.float32)
    h = acc[:, :F1] / (den + 1e-16) + b1_ref[...]
    h = jnp.where(h > 0.0, h, jnp.exp(jnp.minimum(h, 0.0)) - 1.0)
    z = jnp.dot(h, w2_ref[...], preferred_element_type=jnp.float32)
    asd = jnp.dot(z, asd_ref[...], preferred_element_type=jnp.float32)
    o_ref[...] = jnp.concatenate(
        [z, asd, jnp.zeros((z.shape[0], G2 - NCLS - 2), jnp.float32)], axis=1)


_tc2 = pl.pallas_call(
    _tc2_body,
    grid=(10,),
    in_specs=[pl.BlockSpec((NACC // 10, GA1), lambda i: (i, 0)),
              pl.BlockSpec((NACC // 10, GA1), lambda i: (i, 0)),
              pl.BlockSpec((H1, F1), lambda i: (0, 0)),
              pl.BlockSpec((1, F1), lambda i: (0, 0)),
              pl.BlockSpec((F1, NCLS), lambda i: (0, 0)),
              pl.BlockSpec((NCLS, 2), lambda i: (0, 0))],
    out_specs=pl.BlockSpec((NACC // 10, G2), lambda i: (i, 0)),
    out_shape=jax.ShapeDtypeStruct((NACC, G2), jnp.float32),
)


def _tc3_body(q1_ref, q2_ref, b2_ref, o_ref):
    acc = q1_ref[...] + q2_ref[...]
    o = acc[:, :NCLS] / (acc[:, NCLS:NCLS + 1] + 1e-16) + b2_ref[...]
    m = jnp.max(o, axis=1, keepdims=True)
    t = o - m
    o_ref[...] = t - jnp.log(jnp.sum(jnp.exp(t), axis=1, keepdims=True))


_tc3 = pl.pallas_call(
    _tc3_body,
    grid=(10,),
    in_specs=[pl.BlockSpec((NACC // 10, G2), lambda i: (i, 0)),
              pl.BlockSpec((NACC // 10, G2), lambda i: (i, 0)),
              pl.BlockSpec((1, NCLS), lambda i: (0, 0))],
    out_specs=pl.BlockSpec((NACC // 10, NCLS), lambda i: (i, 0)),
    out_shape=jax.ShapeDtypeStruct((NACC, NCLS), jnp.float32),
)


def kernel(x, edge_index, W1, att_src1, att_dst1, b1, W2, att_src2, att_dst2, b2):
    loop = jnp.arange(N, dtype=jnp.int32)
    pad = EP - (edge_index.shape[1] + N)
    src = jnp.concatenate([edge_index[0].astype(jnp.int32), loop,
                           jnp.zeros((pad,), jnp.int32)])
    dst = jnp.concatenate([edge_index[1].astype(jnp.int32), loop,
                           jnp.full((pad,), N, jnp.int32)])
    eye = jnp.eye(H1, dtype=jnp.float32)
    A1s = (att_src1[:, :, None] * eye[:, None, :]).reshape(F1, H1)
    A1d = (att_dst1[:, :, None] * eye[:, None, :]).reshape(F1, H1)
    ho = _tc1(x, W1, A1s, A1d)
    hp = lax.bitcast_convert_type(
        ho[:, :F1].astype(jnp.bfloat16).reshape(N, F1 // 2, 2), jnp.float32)
    hs1 = jnp.concatenate([hp, ho[:, F1:]], axis=1)
    ad1 = jnp.concatenate(
        [ho[:, F1 + H1:],
         jnp.zeros((NACC - N, H1), jnp.float32)], axis=0)
    part1 = _edge_l1(hs1, ad1, src.reshape(EP // 128, 128),
                     dst.reshape(EP // 128, 128))
    perm = jnp.array(PERM, dtype=jnp.int32)
    e8p = (jnp.right_shift(perm, 3)[None, :]
           == jnp.arange(H1, dtype=jnp.int32)[:, None]).astype(jnp.float32)
    b1p = b1[perm]
    W2p = W2[perm, :]
    asd2 = jnp.concatenate([att_src2.T, att_dst2.T], axis=1)
    hs2 = _tc2(part1[0], part1[1], e8p, b1p[None, :], W2p, asd2)
    ad2 = hs2[:, NCLS + 1]
    part2 = _edge_l2(hs2, ad2, src.reshape(EP // 256, 256),
                     dst.reshape(EP // 256, 256))
    out = _tc3(part2[0], part2[1], b2[None, :])
    return out[:N]


# speedup vs baseline: 1.0096x; 1.0096x over previous
import functools

import jax
import jax.numpy as jnp
from jax import lax
from jax.experimental import pallas as pl
from jax.experimental.pallas import tpu as pltpu
from jax.experimental.pallas import tpu_sc as plsc

N = 10000
D = 128
H1, C1 = 8, 8
F1 = H1 * C1
NCLS = 16
NACC = 10240
NC, NS, L = 2, 16, 16
NW = NC * NS
T = 10752
EP = NW * T
RPT = NACC // NS

G1, HC1 = 48, 32
GA1 = 80
G2, HC2 = 32, 16
G2P = 16
GA2 = 48

PERM = []
for _g in range(2):
    PERM += [32 * _g + 2 * _l for _l in range(16)]
    PERM += [32 * _g + 2 * _l + 1 for _l in range(16)]


def _edge_kernel(G, HC, He, ad_full, K, NCH, GA, pack_bf16):
    mesh = plsc.VectorSubcoreMesh(core_axis_name="c", subcore_axis_name="s",
                                  num_cores=NC, num_subcores=NS)
    scratch = [
        pltpu.VMEM_SHARED((NACC, GA), jnp.float32),
        pltpu.VMEM((NCH, K), jnp.int32),
        pltpu.VMEM((NCH, K), jnp.int32),
        pltpu.VMEM((NACC,), jnp.float32) if ad_full
        else pltpu.VMEM((2, K, He), jnp.float32),
        pltpu.VMEM((2, K, G), jnp.float32),
        pltpu.VMEM((K * He,), jnp.float32),
        pltpu.VMEM((2, K, GA), jnp.float32),
        pltpu.SemaphoreType.DMA,
        pltpu.SemaphoreType.DMA,
        pltpu.SemaphoreType.DMA,
        pltpu.SemaphoreType.DMA,
        pltpu.SemaphoreType.DMA,
        pltpu.SemaphoreType.DMA,
    ]

    @functools.partial(
        pl.kernel,
        out_type=jax.ShapeDtypeStruct((NC, NACC, GA), jnp.float32),
        mesh=mesh,
        scratch_types=scratch,
        compiler_params=pltpu.CompilerParams(
            use_tc_tiling_on_sc=False, needs_layout_passes=False),
    )
    def body(hs_hbm, ad_hbm, src_hbm, dst_hbm, out_hbm,
             acc, src_t, dst_t, ad_v, rows_v, w_v, ob_v,
             sg0, sg1, sa0, sa1, ss0, ss1):
        c = lax.axis_index("c")
        s = lax.axis_index("s")
        for zp in (0, 1):
            zrows = ob_v.at[zp]

            def zfill(t, cz, zrows=zrows):
                for j in range(GA // L):
                    zrows[t, pl.ds(j * L, L)] = jnp.zeros((L,), jnp.float32)
                return cz

            lax.fori_loop(0, K, zfill, 0, unroll=4)
        zrows = ob_v.at[0]
        nfull = RPT // K
        for r in range(nfull):
            pltpu.sync_copy(zrows, acc.at[pl.ds(s * RPT + r * K, K)])
        if RPT % K:
            pltpu.sync_copy(zrows.at[pl.ds(0, RPT % K)],
                            acc.at[pl.ds(s * RPT + nfull * K, RPT % K)])
        row0 = (c * NS + s) * NCH
        pltpu.sync_copy(src_hbm.at[pl.ds(row0, NCH)], src_t)
        pltpu.sync_copy(dst_hbm.at[pl.ds(row0, NCH)], dst_t)
        if ad_full:
            pltpu.sync_copy(ad_hbm, ad_v)
        plsc.subcore_barrier()
        iota = lax.iota(jnp.int32, L)
        sg = (sg0, sg1)
        sa = (sa0, sa1)
        ss = (ss0, ss1)

        def g_start(i, p):
            pltpu.async_copy(hs_hbm.at[src_t.at[i]], rows_v.at[p], sg[p])
            if not ad_full:
                pltpu.async_copy(ad_hbm.at[dst_t.at[i]], ad_v.at[p], sa[p])

        def g_wait(p):
            pltpu.make_async_copy(hs_hbm.at[src_t.at[0]], rows_v.at[p],
                                  sg[p]).wait()
            if not ad_full:
                pltpu.make_async_copy(ad_hbm.at[dst_t.at[0]], ad_v.at[p],
                                      sa[p]).wait()

        def s_start(i, p):
            pltpu.async_copy(ob_v.at[p], acc.at[dst_t.at[i]], ss[p], add=True)

        def s_wait(p):
            pltpu.make_async_copy(ob_v.at[p], acc.at[dst_t.at[0]],
                                  ss[p]).wait()

        def compute(i, p):
            rows = rows_v.at[p]
            ob = ob_v.at[p]

            def wpass(t, cw):
                p0 = t * L
                pp = p0 + iota
                if He == 1:
                    k_vec = pp
                    h_vec = jnp.zeros((L,), jnp.int32)
                else:
                    k_vec = jnp.right_shift(pp, 3)
                    h_vec = jnp.bitwise_and(pp, He - 1)
                as_vals = plsc.load_gather(rows, [k_vec, HC + h_vec])
                if ad_full:
                    dvals = dst_t[i, pl.ds(p0, L)]
                    ad_vals = plsc.load_gather(ad_v, [dvals])
                else:
                    ad_vals = plsc.load_gather(ad_v.at[p], [k_vec, h_vec])
                e = as_vals + ad_vals
                e = jnp.where(e >= 0.0, e, 0.2 * e)
                w_v[pl.ds(p0, L)] = jnp.exp(e)
                return cw

            lax.fori_loop(0, K * He // L, wpass, 0, unroll=2)

            if pack_bf16:
                ngrp = max(1, HC // L)

                def mpass(k, cm):
                    wbase = k * He
                    for g in range(ngrp):
                        v = rows[k, pl.ds(g * L, L)]
                        ab = plsc.bitcast(v, jnp.bfloat16)
                        a, b = plsc.unpack(
                            ab, format=plsc.PackFormat.INTERLEAVED)
                        if He == 1:
                            kvec = jnp.broadcast_to(k, (L,)).astype(jnp.int32)
                            wv = plsc.load_gather(w_v, [kvec])
                        else:
                            head = 4 * g + jnp.right_shift(iota, 2)
                            wv = plsc.load_gather(w_v, [wbase + head])
                        nv = min(L, HC - g * L)
                        av, bv = a * wv, b * wv
                        if nv < L:
                            av = jnp.where(iota < nv, av, 0.0)
                            bv = jnp.where(iota < nv, bv, 0.0)
                        ob[k, pl.ds(2 * g * L, L)] = av
                        ob[k, pl.ds((2 * g + 1) * L, L)] = bv
                    widx = wbase + jnp.minimum(iota, He - 1)
                    wvals = plsc.load_gather(w_v, [widx])
                    ob[k, pl.ds(2 * ngrp * L, L)] = jnp.where(
                        iota < He, wvals, 0.0)
                    return cm
            else:
                def mpass(k, cm):
                    wbase = k * He
                    for j in range(GA // L):
                        if (j + 1) * L <= HC:
                            hv = rows[k, pl.ds(j * L, L)]
                            if He == 1:
                                kvec = jnp.broadcast_to(k, (L,)).astype(jnp.int32)
                                wvals = plsc.load_gather(w_v, [kvec])
                            else:
                                head = jnp.right_shift(j * L + iota, 3)
                                wvals = plsc.load_gather(w_v, [wbase + head])
                            ob[k, pl.ds(j * L, L)] = hv * wvals
                        elif j * L == HC:
                            widx = wbase + jnp.minimum(iota, He - 1)
                            wvals = plsc.load_gather(w_v, [widx])
                            ob[k, pl.ds(j * L, L)] = jnp.where(
                                iota < He, wvals, 0.0)
                    return cm

            lax.fori_loop(0, K, mpass, 0, unroll=4)

        g_start(0, 0)
        g_start(1, 1)
        g_wait(0)
        compute(0, 0)
        g_start(2, 0)
        s_start(0, 0)
        g_wait(1)
        compute(1, 1)
        g_start(3, 1)
        s_start(1, 1)

        def step(i2, carry):
            for p in (0, 1):
                i = 2 * i2 + p
                g_wait(p)
                s_wait(p)
                compute(i, p)
                g_start(jnp.minimum(i + 2, NCH - 1), p)
                s_start(i, p)
            return carry

        lax.fori_loop(1, NCH // 2, step, 0)
        g_wait(0)
        g_wait(1)
        s_wait(0)
        s_wait(1)
        plsc.subcore_barrier()
        off = 0
        while off < RPT:
            blk = min(K, RPT - off)
            tmp = ob_v.at[1, pl.ds(0, blk)]
            pltpu.sync_copy(acc.at[pl.ds(s * RPT + off, blk)], tmp)
            pltpu.sync_copy(tmp, out_hbm.at[c, pl.ds(s * RPT + off, blk)])
            off += blk

    return body


_edge_l1 = _edge_kernel(G1, HC1, H1, False, 128, 84, GA1, True)
_edge_l2 = _edge_kernel(G2P, 8, 1, True, 256, 42, GA2, True)


def _tc1_body(x_ref, w_ref, aS_ref, aD_ref, o_ref):
    h = jnp.dot(x_ref[...], w_ref[...], preferred_element_type=jnp.float32)
    aS = jnp.dot(h, aS_ref[...], preferred_element_type=jnp.float32)
    aD = jnp.dot(h, aD_ref[...], preferred_element_type=jnp.float32)
    o_ref[...] = jnp.concatenate([h, aS, aD], axis=1)


_tc1 = pl.pallas_call(
    _tc1_body,
    grid=(10,),
    in_specs=[pl.BlockSpec((N // 10, D), lambda i: (i, 0)),
              pl.BlockSpec((D, F1), lambda i: (0, 0)),
              pl.BlockSpec((F1, H1), lambda i: (0, 0)),
              pl.BlockSpec((F1, H1), lambda i: (0, 0))],
    out_specs=pl.BlockSpec((N // 10, F1 + 2 * H1), lambda i: (i, 0)),
    out_shape=jax.ShapeDtypeStruct((N, F1 + 2 * H1), jnp.float32),
)


def _tc2_body(p1_ref, p2_ref, e8_ref, b1_ref, w2_ref, asd_ref, o_ref):
    acc = p1_ref[...] + p2_ref[...]
    den = jnp.dot(acc[:, F1:F1 + H1], e8_ref[...],
                  preferred_element_type=jnp.float32)
    h = acc[:, :F1] / (den + 1e-16) + b1_ref[...]
    h = jnp.where(h > 0.0, h, jnp.exp(jnp.minimum(h, 0.0)) - 1.0)
    z = jnp.dot(h, w2_ref[...], preferred_element_type=jnp.float32)
    asd = jnp.dot(z, asd_ref[...], preferred_element_type=jnp.float32)
    o_ref[...] = jnp.concatenate(
        [z, asd, jnp.zeros((z.shape[0], G2 - NCLS - 2), jnp.float32)], axis=1)


_tc2 = pl.pallas_call(
    _tc2_body,
    grid=(10,),
    in_specs=[pl.BlockSpec((NACC // 10, GA1), lambda i: (i, 0)),
              pl.BlockSpec((NACC // 10, GA1), lambda i: (i, 0)),
              pl.BlockSpec((H1, F1), lambda i: (0, 0)),
              pl.BlockSpec((1, F1), lambda i: (0, 0)),
              pl.BlockSpec((F1, NCLS), lambda i: (0, 0)),
              pl.BlockSpec((NCLS, 2), lambda i: (0, 0))],
    out_specs=pl.BlockSpec((NACC // 10, G2), lambda i: (i, 0)),
    out_shape=jax.ShapeDtypeStruct((NACC, G2), jnp.float32),
)


def _tc3_body(q1_ref, q2_ref, p_ref, b2_ref, o_ref):
    acc = q1_ref[...] + q2_ref[...]
    msg = jnp.dot(acc[:, :2 * NCLS], p_ref[...],
                  preferred_element_type=jnp.float32)
    o = msg / (acc[:, 2 * NCLS:2 * NCLS + 1] + 1e-16) + b2_ref[...]
    m = jnp.max(o, axis=1, keepdims=True)
    t = o - m
    o_ref[...] = t - jnp.log(jnp.sum(jnp.exp(t), axis=1, keepdims=True))


_tc3 = pl.pallas_call(
    _tc3_body,
    grid=(10,),
    in_specs=[pl.BlockSpec((NACC // 10, GA2), lambda i: (i, 0)),
              pl.BlockSpec((NACC // 10, GA2), lambda i: (i, 0)),
              pl.BlockSpec((2 * NCLS, NCLS), lambda i: (0, 0)),
              pl.BlockSpec((1, NCLS), lambda i: (0, 0))],
    out_specs=pl.BlockSpec((NACC // 10, NCLS), lambda i: (i, 0)),
    out_shape=jax.ShapeDtypeStruct((NACC, NCLS), jnp.float32),
)


def kernel(x, edge_index, W1, att_src1, att_dst1, b1, W2, att_src2, att_dst2, b2):
    loop = jnp.arange(N, dtype=jnp.int32)
    pad = EP - (edge_index.shape[1] + N)
    src = jnp.concatenate([edge_index[0].astype(jnp.int32), loop,
                           jnp.zeros((pad,), jnp.int32)])
    dst = jnp.concatenate([edge_index[1].astype(jnp.int32), loop,
                           jnp.full((pad,), N, jnp.int32)])
    eye = jnp.eye(H1, dtype=jnp.float32)
    A1s = (att_src1[:, :, None] * eye[:, None, :]).reshape(F1, H1)
    A1d = (att_dst1[:, :, None] * eye[:, None, :]).reshape(F1, H1)
    ho = _tc1(x, W1, A1s, A1d)
    hp = lax.bitcast_convert_type(
        ho[:, :F1].astype(jnp.bfloat16).reshape(N, F1 // 2, 2), jnp.float32)
    hs1 = jnp.concatenate([hp, ho[:, F1:]], axis=1)
    ad1 = jnp.concatenate(
        [ho[:, F1 + H1:],
         jnp.zeros((NACC - N, H1), jnp.float32)], axis=0)
    part1 = _edge_l1(hs1, ad1, src.reshape(EP // 128, 128),
                     dst.reshape(EP // 128, 128))
    perm = jnp.array(PERM, dtype=jnp.int32)
    e8p = (jnp.right_shift(perm, 3)[None, :]
           == jnp.arange(H1, dtype=jnp.int32)[:, None]).astype(jnp.float32)
    b1p = b1[perm]
    W2p = W2[perm, :]
    asd2 = jnp.concatenate([att_src2.T, att_dst2.T], axis=1)
    hs2 = _tc2(part1[0], part1[1], e8p, b1p[None, :], W2p, asd2)
    ad2 = hs2[:, NCLS + 1]
    zp = lax.bitcast_convert_type(
        hs2[:, :NCLS].astype(jnp.bfloat16).reshape(NACC, NCLS // 2, 2),
        jnp.float32)
    hs2p = jnp.concatenate(
        [zp, hs2[:, NCLS:NCLS + 2],
         jnp.zeros((NACC, G2P - NCLS // 2 - 2), jnp.float32)], axis=1)
    part2 = _edge_l2(hs2p, ad2, src.reshape(EP // 256, 256),
                     dst.reshape(EP // 256, 256))
    p32 = jnp.zeros((2 * NCLS, NCLS), jnp.float32)
    lidx = jnp.arange(NCLS // 2)
    p32 = p32.at[lidx, 2 * lidx].set(1.0)
    p32 = p32.at[NCLS + lidx, 2 * lidx + 1].set(1.0)
    out = _tc3(part2[0], part2[1], p32, b2[None, :])
    return out[:N]
